# baseline (speedup 1.0000x reference)
import functools

import jax
import jax.numpy as jnp
from jax import lax
from jax.experimental import pallas as pl
from jax.experimental.pallas import tpu as pltpu
from jax.experimental.pallas import tpu_sc as plsc

N_ROWS = 16384
IN_CH = 2048
L = 16
NC = 2
NS = 16
NW = NC * NS
ROWS_PER_W = N_ROWS // NW
CHUNK = 8
CE = CHUNK * IN_CH
N_STEPS = ROWS_PER_W // CHUNK
NG = IN_CH // L
UNROLL = 8
NBI = 4
NBO = 2


def _permute_body(x_hbm, p_hbm, out_hbm, p_v, *rest):
    xins = rest[:NBI]
    xouts = rest[NBI:NBI + NBO]
    sis = rest[NBI + NBO:2 * NBI + NBO]
    sos = rest[2 * NBI + NBO:2 * NBI + 2 * NBO]

    wid = lax.axis_index("s") * NC + lax.axis_index("c")
    row0 = wid * ROWS_PER_W
    pltpu.sync_copy(p_hbm, p_v)

    def start_in(c, b):
        src = x_hbm.at[pl.ds(row0 + c * CHUNK, CHUNK), :]
        pltpu.async_copy(src, xins[b], sis[b])

    def start_out(c, b):
        dst = out_hbm.at[pl.ds(row0 + c * CHUNK, CHUNK), :]
        pltpu.async_copy(xouts[b], dst, sos[b])

    def wait_in(b):
        pltpu.make_async_copy(x_hbm.at[pl.ds(row0, CHUNK), :], xins[b], sis[b]).wait()

    def wait_out(b):
        pltpu.make_async_copy(xouts[b], out_hbm.at[pl.ds(row0, CHUNK), :], sos[b]).wait()

    for b in range(NBI):
        start_in(b, b)

    def chunk_body(c, bi, bo):
        @pl.when(c >= NBO)
        def _():
            wait_out(bo)
        wait_in(bi)

        @plsc.parallel_loop(0, NG, 1, unroll=UNROLL)
        def _(g):
            off = pl.multiple_of(g * L, L)
            idx = p_v[pl.ds(off, L)]
            for r in range(CHUNK):
                row_idx = jnp.full((L,), r, jnp.int32)
                v = plsc.load_gather(xins[bi], [row_idx, idx])
                xouts[bo][r, pl.ds(off, L)] = v

        start_out(c, bo)

        @pl.when(c + NBI < N_STEPS)
        def _():
            start_in(c + NBI, bi)

    def ring_body(i, carry):
        for j in range(NBI):
            c = NBI * i + j
            chunk_body(c, j, j % NBO)
        return carry

    lax.fori_loop(0, N_STEPS // NBI, ring_body, 0)
    for b in range(NBO):
        wait_out(b)


@jax.jit
def _permute(x, p):
    mesh = plsc.VectorSubcoreMesh(core_axis_name="c", subcore_axis_name="s")
    return pl.kernel(
        _permute_body,
        out_type=jax.ShapeDtypeStruct((N_ROWS, IN_CH), jnp.float32),
        mesh=mesh,
        scratch_types=(
            [pltpu.VMEM((IN_CH,), jnp.int32)]
            + [pltpu.VMEM((CHUNK, IN_CH), jnp.float32) for _ in range(NBI + NBO)]
            + [pltpu.SemaphoreType.DMA for _ in range(NBI + NBO)]
        ),
        compiler_params=pltpu.CompilerParams(needs_layout_passes=False),
    )(x, p)


def kernel(x, p):
    out = _permute(x, p.astype(jnp.int32))
    return (out, 0)
